# baseline (speedup 1.0000x reference)
import functools

import jax
import jax.numpy as jnp
from jax import lax
from jax.experimental import pallas as pl
from jax.experimental.pallas import tpu as pltpu
from jax.experimental.pallas import tpu_sc as plsc

B = 16
N_POS = 3072
LEVELS = 256
D = 4096

D_SC = 512
D_TC = D - D_SC

NC = 2
NS = 16
NW = NC * NS
DW = D_SC // NW
PCHUNK = 256
NCHUNK = N_POS // PCHUNK

D_TILE = 512


def _sc_body(xt_hbm, pos_hbm, tab_hbm, out_hbm,
             tab_v, pos_v0, pos_v1, xt_c, acc_v, sem0, sem1):
    c = lax.axis_index("c")
    s = lax.axis_index("s")
    wid = s * NC + c
    dblk = D_TC + (wid // 8) * 128
    soff = (wid % 8) * DW

    pltpu.sync_copy(tab_hbm.at[wid], tab_v)

    zero = jnp.zeros((16,), jnp.float32)
    for b in range(B):
        acc_v[pl.ds(b * DW, DW)] = zero

    base = tab_v[pl.ds(0, DW)]

    def t_body(l, cnt):
        row = tab_v[pl.ds(l * DW, DW)]
        return cnt + jnp.where(row == base, 1, 0).astype(jnp.int32)

    tflip = lax.fori_loop(0, LEVELS, t_body, jnp.zeros((16,), jnp.int32))

    bufs = [pos_v0, pos_v1]
    sems = [sem0, sem1]
    pending = pltpu.async_copy(
        pos_hbm.at[pl.ds(0, PCHUNK), pl.ds(dblk, 128)], bufs[0], sems[0])
    for ci in range(NCHUNK):
        p0 = ci * PCHUNK
        pltpu.sync_copy(xt_hbm.at[pl.ds(p0, PCHUNK), :], xt_c)
        pending.wait()
        if ci + 1 < NCHUNK:
            pending = pltpu.async_copy(
                pos_hbm.at[pl.ds((ci + 1) * PCHUNK, PCHUNK), pl.ds(dblk, 128)],
                bufs[(ci + 1) % 2], sems[(ci + 1) % 2])
        pos_v = bufs[ci % 2]

        def p_body(pj, accs, _pos_v=pos_v):
            idxb = (xt_c[pj, :] * float(LEVELS - 1)).astype(jnp.int32)
            pv = _pos_v[pj, pl.ds(soff, DW)]
            pvb = pv * base
            nvb = -pvb
            out = []
            for b in range(B):
                r = jnp.broadcast_to(idxb[b], (16,))
                out.append(accs[b] + jnp.where(r < tflip, pvb, nvb))
            return tuple(out)

        accs = lax.fori_loop(0, PCHUNK, p_body,
                             tuple(zero for _ in range(B)))
        for b in range(B):
            plsc.addupdate(acc_v.at[pl.ds(b * DW, DW)], accs[b])

    for b in range(B):
        sl = pl.ds(b * DW, DW)
        v = acc_v[sl]
        acc_v[sl] = jnp.where(v > 0.0, 1.0, -1.0)
    pltpu.sync_copy(acc_v, out_hbm.at[wid])


def _fire_tc_kernel(xf_ref, pos_ref, tab_ref, out_ref, oh_ref):
    @pl.when(pl.program_id(0) == 0)
    def _build_onehot():
        for b in range(B):
            idx = (xf_ref[b:b + 1, :] * float(LEVELS - 1)).astype(jnp.int32)
            lv = jax.lax.broadcasted_iota(jnp.int32, (LEVELS, N_POS), 0)
            oh_ref[pl.ds(b * LEVELS, LEVELS), :] = (lv == idx).astype(jnp.int8)

    pos_i8 = pos_ref[:, :].astype(jnp.int8)
    q = jnp.dot(oh_ref[:, :], pos_i8, preferred_element_type=jnp.int32
                ).astype(jnp.float32)
    tab = tab_ref[:, :]
    for b in range(B):
        acc = jnp.sum(tab * q[b * LEVELS:(b + 1) * LEVELS, :], axis=0)
        out_ref[b, :] = jnp.where(acc > 0.0, 1.0, -1.0)


@jax.jit
def kernel(x, position, value_table):
    xf = x.reshape(B, N_POS)

    tc_out = pl.pallas_call(
        _fire_tc_kernel,
        grid=(D_TC // D_TILE,),
        in_specs=[
            pl.BlockSpec((B, N_POS), lambda i: (0, 0)),
            pl.BlockSpec((N_POS, D_TILE), lambda i: (0, i)),
            pl.BlockSpec((LEVELS, D_TILE), lambda i: (0, i)),
        ],
        out_specs=pl.BlockSpec((B, D_TILE), lambda i: (0, i)),
        out_shape=jax.ShapeDtypeStruct((B, D_TC), jnp.float32),
        scratch_shapes=[pltpu.VMEM((B * LEVELS, N_POS), jnp.int8)],
    )(xf, position, value_table)

    xt = xf.T
    tab_r = (value_table[:, D_TC:]
             .reshape(LEVELS, NW, DW).transpose(1, 0, 2).reshape(NW, LEVELS * DW))
    sc_call = functools.partial(
        pl.kernel,
        out_type=jax.ShapeDtypeStruct((NW, B * DW), jnp.float32),
        mesh=plsc.VectorSubcoreMesh(core_axis_name="c", subcore_axis_name="s"),
        compiler_params=pltpu.CompilerParams(needs_layout_passes=False),
        scratch_types=[
            pltpu.VMEM((LEVELS * DW,), jnp.float32),
            pltpu.VMEM((PCHUNK, 128), jnp.float32),
            pltpu.VMEM((PCHUNK, 128), jnp.float32),
            pltpu.VMEM((PCHUNK, B), jnp.float32),
            pltpu.VMEM((B * DW,), jnp.float32),
            pltpu.SemaphoreType.DMA,
            pltpu.SemaphoreType.DMA,
        ],
    )(_sc_body)
    sc_r = sc_call(xt, position, tab_r)
    sc_out = sc_r.reshape(NW, B, DW).transpose(1, 0, 2).reshape(B, D_SC)

    return jnp.concatenate([tc_out, sc_out], axis=1)


# speedup vs baseline: 1.0474x; 1.0474x over previous
import functools

import jax
import jax.numpy as jnp
from jax import lax
from jax.experimental import pallas as pl
from jax.experimental.pallas import tpu as pltpu
from jax.experimental.pallas import tpu_sc as plsc

B = 16
N_POS = 3072
LEVELS = 256
D = 4096

D_SC = 512
D_TC = D - D_SC

NC = 2
NS = 16
NW = NC * NS
DW = D_SC // NW
PCHUNK = 256
NCHUNK = N_POS // PCHUNK

D_TILE = 512


def _sc_body(xt_hbm, pos_hbm, tab_hbm, out_hbm,
             tab_v, pos_v0, pos_v1, xt_c, acc_v, sem0, sem1):
    c = lax.axis_index("c")
    s = lax.axis_index("s")
    wid = s * NC + c
    dblk = D_TC + (wid // 8) * 128
    soff = (wid % 8) * DW

    pltpu.sync_copy(tab_hbm.at[wid], tab_v)

    zero = jnp.zeros((16,), jnp.float32)
    for b in range(B):
        acc_v[pl.ds(b * DW, DW)] = zero

    base = tab_v[pl.ds(0, DW)]

    def t_body(l, cnt):
        row = tab_v[pl.ds(l * DW, DW)]
        return cnt + jnp.where(row == base, 1, 0).astype(jnp.int32)

    tflip = lax.fori_loop(0, LEVELS, t_body, jnp.zeros((16,), jnp.int32))

    bufs = [pos_v0, pos_v1]
    sems = [sem0, sem1]
    pending = pltpu.async_copy(
        pos_hbm.at[pl.ds(0, PCHUNK), pl.ds(dblk, 128)], bufs[0], sems[0])
    for ci in range(NCHUNK):
        p0 = ci * PCHUNK
        pltpu.sync_copy(xt_hbm.at[pl.ds(p0, PCHUNK), :], xt_c)
        pending.wait()
        if ci + 1 < NCHUNK:
            pending = pltpu.async_copy(
                pos_hbm.at[pl.ds((ci + 1) * PCHUNK, PCHUNK), pl.ds(dblk, 128)],
                bufs[(ci + 1) % 2], sems[(ci + 1) % 2])
        pos_v = bufs[ci % 2]

        def p_body(pj, accs, _pos_v=pos_v):
            idxb = (xt_c[pj, :] * float(LEVELS - 1)).astype(jnp.int32)
            pv = _pos_v[pj, pl.ds(soff, DW)]
            pvb = pv * base
            nvb = -pvb
            out = []
            for b in range(B):
                r = jnp.broadcast_to(idxb[b], (16,))
                out.append(accs[b] + jnp.where(r < tflip, pvb, nvb))
            return tuple(out)

        accs = lax.fori_loop(0, PCHUNK, p_body,
                             tuple(zero for _ in range(B)))
        for b in range(B):
            plsc.addupdate(acc_v.at[pl.ds(b * DW, DW)], accs[b])

    for b in range(B):
        sl = pl.ds(b * DW, DW)
        v = acc_v[sl]
        acc_v[sl] = jnp.where(v > 0.0, 1.0, -1.0)
    pltpu.sync_copy(acc_v, out_hbm.at[wid])


def _fire_tc_kernel(xf_ref, pos_ref, tab_ref, out_ref, oh_ref):
    @pl.when(pl.program_id(0) == 0)
    def _build_onehot():
        for b in range(B):
            idx = (xf_ref[b:b + 1, :] * float(LEVELS - 1)).astype(jnp.int32)
            lv = jax.lax.broadcasted_iota(jnp.int32, (LEVELS, N_POS), 0)
            oh_ref[pl.ds(b * LEVELS, LEVELS), :] = (lv == idx).astype(jnp.bfloat16)

    pos_bf = pos_ref[:, :].astype(jnp.bfloat16)
    q = jnp.dot(oh_ref[:, :], pos_bf, preferred_element_type=jnp.float32)
    tab = tab_ref[:, :]
    for b in range(B):
        acc = jnp.sum(tab * q[b * LEVELS:(b + 1) * LEVELS, :], axis=0)
        out_ref[b, :] = jnp.where(acc > 0.0, 1.0, -1.0)


@jax.jit
def kernel(x, position, value_table):
    xf = x.reshape(B, N_POS)

    tc_out = pl.pallas_call(
        _fire_tc_kernel,
        grid=(D_TC // D_TILE,),
        in_specs=[
            pl.BlockSpec((B, N_POS), lambda i: (0, 0)),
            pl.BlockSpec((N_POS, D_TILE), lambda i: (0, i)),
            pl.BlockSpec((LEVELS, D_TILE), lambda i: (0, i)),
        ],
        out_specs=pl.BlockSpec((B, D_TILE), lambda i: (0, i)),
        out_shape=jax.ShapeDtypeStruct((B, D_TC), jnp.float32),
        scratch_shapes=[pltpu.VMEM((B * LEVELS, N_POS), jnp.bfloat16)],
    )(xf, position, value_table)

    xt = xf.T
    tab_r = (value_table[:, D_TC:]
             .reshape(LEVELS, NW, DW).transpose(1, 0, 2).reshape(NW, LEVELS * DW))
    sc_call = functools.partial(
        pl.kernel,
        out_type=jax.ShapeDtypeStruct((NW, B * DW), jnp.float32),
        mesh=plsc.VectorSubcoreMesh(core_axis_name="c", subcore_axis_name="s"),
        compiler_params=pltpu.CompilerParams(needs_layout_passes=False),
        scratch_types=[
            pltpu.VMEM((LEVELS * DW,), jnp.float32),
            pltpu.VMEM((PCHUNK, 128), jnp.float32),
            pltpu.VMEM((PCHUNK, 128), jnp.float32),
            pltpu.VMEM((PCHUNK, B), jnp.float32),
            pltpu.VMEM((B * DW,), jnp.float32),
            pltpu.SemaphoreType.DMA,
            pltpu.SemaphoreType.DMA,
        ],
    )(_sc_body)
    sc_r = sc_call(xt, position, tab_r)
    sc_out = sc_r.reshape(NW, B, DW).transpose(1, 0, 2).reshape(B, D_SC)

    return jnp.concatenate([tc_out, sc_out], axis=1)
